# baseline (speedup 1.0000x reference)
import jax
import jax.numpy as jnp
from jax import lax
from jax.experimental import pallas as pl
from jax.experimental.pallas import tpu as pltpu
from jax.experimental.pallas import tpu_sc as plsc

NUM_EMBEDDINGS = 8192
EMBEDDING_DIM = 64
BATCH = 32
TOKENS = 1024

_NC = 2
_NS = 16
_VH = NUM_EMBEDDINGS // 128


def _gather_body(table_hbm, idx_hbm, out_hbm, idx_v, row0, row1, o0, o1,
                 tsem, isem, ssem0, ssem1):
    wid = lax.axis_index("s") * _NC + lax.axis_index("c")
    f0 = wid * 2
    f1 = f0 + 1
    fh0 = f0 // 8
    fl0 = f0 % 8
    fh1 = f1 // 8
    fl1 = f1 % 8

    t0 = pltpu.async_copy(table_hbm.at[fh0, :, fl0], row0, tsem)
    t1 = pltpu.async_copy(table_hbm.at[fh1, :, fl1], row1, tsem)
    idx_cp = [
        pltpu.async_copy(idx_hbm.at[q], idx_v.at[q], isem) for q in range(4)
    ]
    t0.wait()
    t1.wait()

    stores = []
    for q in range(4):
        idx_cp[q].wait()

        @plsc.parallel_loop(0, 64, step=1, unroll=2)
        def _slab(g):
            rl = g // 8
            ch = g % 8
            b = q * 8 + rl
            for cl0 in range(8):
                v = idx_v[q, ch, rl, pl.ds(cl0 * 16, 16)]
                vh = v >> 7
                vl = v & 127
                o0[b, ch, pl.ds(cl0 * 16, 16)] = plsc.load_gather(row0, [vh, vl])
                o1[b, ch, pl.ds(cl0 * 16, 16)] = plsc.load_gather(row1, [vh, vl])

        stores.append(
            pltpu.async_copy(
                o0.at[pl.ds(q * 8, 8)],
                out_hbm.at[pl.ds(q * 8, 8), fh0, :, fl0, :],
                ssem0,
            )
        )
        stores.append(
            pltpu.async_copy(
                o1.at[pl.ds(q * 8, 8)],
                out_hbm.at[pl.ds(q * 8, 8), fh1, :, fl1, :],
                ssem1,
            )
        )
    for s in stores:
        s.wait()


_gather_call = pl.kernel(
    _gather_body,
    out_type=jax.ShapeDtypeStruct((BATCH, 8, 8, 8, 128), jnp.float32),
    mesh=plsc.VectorSubcoreMesh(core_axis_name="c", subcore_axis_name="s"),
    scratch_types=[
        pltpu.VMEM((4, 8, 8, 128), jnp.int32),
        pltpu.VMEM((_VH, 128), jnp.float32),
        pltpu.VMEM((_VH, 128), jnp.float32),
        pltpu.VMEM((BATCH, 8, 128), jnp.float32),
        pltpu.VMEM((BATCH, 8, 128), jnp.float32),
        pltpu.SemaphoreType.DMA,
        pltpu.SemaphoreType.DMA,
        pltpu.SemaphoreType.DMA,
        pltpu.SemaphoreType.DMA,
    ],
    compiler_params=pltpu.CompilerParams(
        use_tc_tiling_on_sc=False, needs_layout_passes=False
    ),
)


@jax.jit
def kernel(indices, embeddings):
    idx_t = (
        jnp.asarray(indices, jnp.int32)
        .reshape(4, 8, 8, 128)
        .transpose(0, 2, 1, 3)
    )
    table4 = embeddings.T.reshape(8, 8, _VH, 128).transpose(0, 2, 1, 3)
    out5 = _gather_call(table4, idx_t)
    return out5.transpose(0, 2, 4, 1, 3).reshape(BATCH, TOKENS, EMBEDDING_DIM)
